# baseline (speedup 1.0000x reference)
import functools

import jax
import jax.numpy as jnp
from jax import lax
from jax.experimental import pallas as pl
from jax.experimental.pallas import tpu as pltpu
from jax.experimental.pallas import tpu_sc as plsc

N_F = 26
V = 100000
E = 32
EP = 128
B = 16384

NC = 2
NS = 16
NW = NC * NS
BPW = B // NW
HB = BPW // 2
CHUNK = 128
NCH = HB // CHUNK
L = 16
NSTEP = N_F * 2


def _body(idx_hbm, tab_hbm, out_hbm, idx_v, rows_v, t_v, gsem, wsem):
    wid = lax.axis_index("s") * NC + lax.axis_index("c")
    base = wid * BPW
    pltpu.sync_copy(idx_hbm.at[:, pl.ds(base, BPW)], idx_v)

    def g_start(s, p):
        f = s // 2
        h = lax.rem(s, 2)
        for c in range(NCH):
            pltpu.make_async_copy(
                tab_hbm.at[idx_v.at[f, pl.ds(h * HB + c * CHUNK, CHUNK)]],
                rows_v.at[p, pl.ds(c * CHUNK, CHUNK), :],
                gsem,
            ).start()

    def g_wait(p):
        pltpu.make_async_copy(
            tab_hbm.at[pl.ds(0, HB)], rows_v.at[p], gsem
        ).wait()

    def transpose(p, q):
        def erow(e, carry):
            for k in range(HB // L):
                bvec = jax.lax.iota(jnp.int32, L) + k * L
                evec = jnp.zeros((L,), jnp.int32) + e
                t_v[q, e, pl.ds(k * L, L)] = plsc.load_gather(
                    rows_v.at[p], [bvec, evec]
                )
            return carry

        lax.fori_loop(0, E, erow, 0)

    def w_desc(s, q):
        f = s // 2
        h = lax.rem(s, 2)
        return pltpu.make_async_copy(
            t_v.at[q],
            out_hbm.at[pl.ds(f * E, E), pl.ds(base + h * HB, HB)],
            wsem,
        )

    g_start(0, 0)
    g_wait(0)
    g_start(1, 1)
    transpose(0, 0)
    w_desc(0, 0).start()

    def step(s, carry):
        p = lax.rem(s, 2)
        q = 1 - p
        g_wait(p)
        g_start(s + 1, q)
        w_desc(s - 1, q).wait()
        transpose(p, p)
        w_desc(s, p).start()
        return carry

    lax.fori_loop(1, NSTEP - 1, step, 0)

    p = (NSTEP - 1) % 2
    q = 1 - p
    g_wait(p)
    w_desc(NSTEP - 2, q).wait()
    transpose(p, p)
    w_desc(NSTEP - 1, p).start()
    w_desc(NSTEP - 1, p).wait()


@functools.partial(
    pl.kernel,
    out_type=jax.ShapeDtypeStruct((N_F * E, B), jnp.float32),
    mesh=plsc.VectorSubcoreMesh(core_axis_name="c", subcore_axis_name="s"),
    compiler_params=pltpu.CompilerParams(
        use_tc_tiling_on_sc=True, needs_layout_passes=False
    ),
    scratch_types=[
        pltpu.VMEM((N_F, BPW), jnp.int32),
        pltpu.VMEM((2, HB, EP), jnp.float32),
        pltpu.VMEM((2, E, HB), jnp.float32),
        pltpu.SemaphoreType.DMA,
        pltpu.SemaphoreType.DMA,
    ],
)
def _gather_kernel(idx_hbm, tab_hbm, out_hbm, idx_v, rows_v, t_v, gsem, wsem):
    _body(idx_hbm, tab_hbm, out_hbm, idx_v, rows_v, t_v, gsem, wsem)


def kernel(families, tables):
    fam = families.astype(jnp.int32)
    offs = (jnp.arange(N_F, dtype=jnp.int32) * V)[:, None]
    idx2 = fam + offs
    tab = jnp.pad(tables.reshape(N_F * V, E), ((0, 0), (0, EP - E)))
    outT = _gather_kernel(idx2, tab)
    return outT.T
